# baseline (speedup 1.0000x reference)
import functools

import jax
import jax.numpy as jnp
from jax import lax
from jax.experimental import pallas as pl
from jax.experimental.pallas import tpu as pltpu
from jax.experimental.pallas import tpu_sc as plsc

_PROPOSALS = 1000
_NMS_T = 0.7
_PRE = 6000
_ROWS = 8
_COLS = 768

_B = 2
_N = 261888
_NPAD = 262144
_CHUNK = 16384
_NT = 16
_SELO = 6400
_CAP = 6016
_PERT = _SELO // _NT


def _select_kernel(
    probs_hbm, bbox_hbm, anch_hbm,
    scr_hbm, idx_hbm, dlt_hbm, anc_hbm,
    stage, ubuf, hist, merged, totals, abv_idx, abv_scr, eq_idx, eqscr,
    posbuf, posd, clbuf, cnt_pub, cnts_rd, initf, initi, idx4, rowd, rowa,
    spm_hist, spm_cnt, sem1, sem2,
):
    b = lax.axis_index("c")
    w = lax.axis_index("s")
    iota = lax.broadcasted_iota(jnp.int32, (16,), 0)
    zrow = jnp.zeros((16,), jnp.int32)

    neg = jnp.full((16,), -2e9, jnp.float32)

    def init_body(j, _):
        initf[pl.ds(j * 16, 16)] = neg
        initi[pl.ds(j * 16, 16)] = zrow
        return 0

    lax.fori_loop(0, _PERT // 16, init_body, 0)
    obase = b * _SELO + w * _PERT
    pltpu.sync_copy(initf, scr_hbm.at[pl.ds(obase, _PERT)])
    pltpu.sync_copy(initi, idx_hbm.at[pl.ds(obase, _PERT)])

    start = b * (_NPAD * 2) + w * (_CHUNK * 2)
    pltpu.sync_copy(probs_hbm.at[pl.ds(start, _CHUNK * 2)], stage)
    io2 = iota * 2 + 1

    def deint(i, _):
        gi = plsc.load_gather(stage, [i * 32 + io2])
        ubuf[pl.ds(i * 16, 16)] = plsc.bitcast(gi, jnp.uint32)
        return 0

    lax.fori_loop(0, _CHUNK // 16, deint, 0)

    ones = jnp.ones((16,), jnp.int32)

    def hist_pass(shift, pshift, prefix, k):
        def z(j, _):
            hist[pl.ds(j * 16, 16)] = zrow
            return 0

        lax.fori_loop(0, 1024, z, 0)

        def scan(i, _):
            u = ubuf[pl.ds(i * 16, 16)]
            pred = jnp.right_shift(u, jnp.uint32(pshift)) == prefix
            digit = (jnp.right_shift(u, jnp.uint32(shift)) & jnp.uint32(1023)).astype(jnp.int32)
            plsc.addupdate_scatter(hist, [iota * 1024 + digit], ones, mask=pred)
            return 0

        lax.fori_loop(0, _CHUNK // 16, scan, 0)

        def mg(j, _):
            def inner(c, acc):
                return acc + hist[pl.ds(c * 1024 + j * 16, 16)]
            acc = lax.fori_loop(0, 16, inner, zrow)
            merged[pl.ds(j * 16, 16)] = acc
            return 0

        lax.fori_loop(0, 64, mg, 0)
        pltpu.sync_copy(merged, spm_hist.at[pl.ds(w * 1024, 1024)])
        plsc.subcore_barrier()
        pltpu.sync_copy(spm_hist, hist)
        plsc.subcore_barrier()

        def mg2(j, _):
            def inner(c, acc):
                return acc + hist[pl.ds(c * 1024 + j * 16, 16)]
            acc = lax.fori_loop(0, 16, inner, zrow)
            totals[pl.ds(j * 16, 16)] = acc
            return 0

        lax.fori_loop(0, 64, mg2, 0)

        def find(jj, carry):
            acc, found, bin_, rem = carry
            j = 63 - jj
            chunk = totals[pl.ds(j * 16, 16)]
            csum = jnp.sum(chunk)
            hit = jnp.logical_and(found == 0, acc + csum >= k)
            rv = lax.rev(chunk, (0,))
            cum = plsc.cumsum(rv)
            need = k - acc
            pos = jnp.sum(jnp.where(cum < need, 1, 0).astype(jnp.int32))
            cnt_at = jnp.sum(jnp.where(iota == pos, rv, 0))
            cum_at = jnp.sum(jnp.where(iota == pos, cum, 0))
            bin_new = j * 16 + 15 - pos
            rem_new = need - (cum_at - cnt_at)
            return (
                acc + jnp.where(hit, 0, csum),
                jnp.where(hit, jnp.int32(1), found),
                jnp.where(hit, bin_new, bin_),
                jnp.where(hit, rem_new, rem),
            )

        zero = jnp.int32(0)
        _, _, bin_, rem = lax.fori_loop(0, 64, find, (zero, zero, zero, zero))
        return bin_, rem

    b0, k1 = hist_pass(20, 30, jnp.uint32(0), jnp.int32(_PRE))
    b1, k2 = hist_pass(10, 20, b0.astype(jnp.uint32), k1)
    pref2 = ((b0 << 10) | b1).astype(jnp.uint32)
    b2, k_eq = hist_pass(0, 10, pref2, k2)
    t_bits = ((b0 << 20) | (b1 << 10) | b2).astype(jnp.uint32)
    t_vec = plsc.bitcast(jnp.zeros((16,), jnp.uint32) + t_bits, jnp.float32)

    def fill_eqscr(j, _):
        eqscr[pl.ds(j * 16, 16)] = t_vec
        return 0

    lax.fori_loop(0, 8, fill_eqscr, 0)

    base_g = w * _CHUNK

    def sel(i, carry):
        ac, ec = carry
        u = ubuf[pl.ds(i * 16, 16)]
        gidx = base_g + i * 16 + iota
        above = u > t_bits
        eq = u == t_bits
        am = above.astype(jnp.int32)
        em = eq.astype(jnp.int32)
        apos = ac + plsc.cumsum(am) - 1
        epos = ec + plsc.cumsum(em) - 1
        ma = jnp.logical_and(above, apos < _CAP)
        me = jnp.logical_and(eq, epos < _CAP)
        plsc.store_scatter(abv_idx, [apos], gidx, mask=ma)
        plsc.store_scatter(abv_scr, [apos], plsc.bitcast(u, jnp.float32), mask=ma)
        plsc.store_scatter(eq_idx, [epos], gidx, mask=me)
        return ac + jnp.sum(am), ec + jnp.sum(em)

    ac, ec = lax.fori_loop(0, _CHUNK // 16, sel, (jnp.int32(0), jnp.int32(0)))

    cnt_pub[pl.ds(0, 16)] = jnp.where(
        iota == 0, ac, jnp.where(iota == 1, ec, jnp.int32(0))
    )
    pltpu.sync_copy(cnt_pub, spm_cnt.at[pl.ds(w * 16, 16)])
    plsc.subcore_barrier()
    pltpu.sync_copy(spm_cnt, cnts_rd)
    plsc.subcore_barrier()

    def pref(x, carry):
        ap, ep, at = carry
        row = cnts_rd[pl.ds(x * 16, 16)]
        a_x = jnp.sum(jnp.where(iota == 0, row, 0))
        e_x = jnp.sum(jnp.where(iota == 1, row, 0))
        lt = (x < w).astype(jnp.int32)
        return ap + a_x * lt, ep + e_x * lt, at + a_x

    ap, ep, at = lax.fori_loop(
        0, 16, pref, (jnp.int32(0), jnp.int32(0), jnp.int32(0))
    )
    take = jnp.clip(k_eq - ep, 0, ec)
    a_base = b * _SELO + ap
    e_base = b * _SELO + at + jnp.minimum(ep, k_eq)
    trash = b * _SELO + 6144 + w * 16

    def zf(j, _):
        initf[pl.ds(j * 16, 16)] = jnp.zeros((16,), jnp.float32)
        return 0

    lax.fori_loop(0, _PERT // 16, zf, 0)
    for c in range(4):
        pltpu.sync_copy(initf, dlt_hbm.at[pl.ds(c * (_B * _SELO) + obase, _PERT)])
        pltpu.sync_copy(initf, anc_hbm.at[pl.ds(c * (_B * _SELO) + obase, _PERT)])

    def emit(src_idx, cnt, base, scr_src_fn):
        def chunk(c5, _):
            def fill(j, _):
                r = c5 * 128 + j * 16 + iota
                posbuf[pl.ds(j * 16, 16)] = jnp.where(r < cnt, base + r, trash)
                v = src_idx[pl.ds(c5 * 128 + j * 16, 16)]
                clbuf[pl.ds(j * 16, 16)] = (
                    jnp.minimum(jnp.maximum(v, 0), _N - 1) + b * _N
                ) * 4
                return 0

            lax.fori_loop(0, 8, fill, 0)
            d1 = pltpu.async_copy(
                src_idx.at[pl.ds(c5 * 128, 128)], idx_hbm.at[posbuf], sem1
            )
            d2 = pltpu.async_copy(scr_src_fn(c5), scr_hbm.at[posbuf], sem2)
            d1.wait()
            d2.wait()
            for c in range(4):
                def mk(j, _):
                    idx4[pl.ds(j * 16, 16)] = clbuf[pl.ds(j * 16, 16)] + c
                    posd[pl.ds(j * 16, 16)] = posbuf[pl.ds(j * 16, 16)] + c * (_B * _SELO)
                    return 0

                lax.fori_loop(0, 8, mk, 0)
                g1 = pltpu.async_copy(bbox_hbm.at[idx4], rowd, sem1)
                g2 = pltpu.async_copy(anch_hbm.at[idx4], rowa, sem2)
                g1.wait()
                g2.wait()
                s1 = pltpu.async_copy(rowd, dlt_hbm.at[posd], sem1)
                s2 = pltpu.async_copy(rowa, anc_hbm.at[posd], sem2)
                s1.wait()
                s2.wait()
            return 0

        lax.fori_loop(0, (cnt + 127) // 128, chunk, 0)

    emit(abv_idx, ac, a_base, lambda c5: abv_scr.at[pl.ds(c5 * 128, 128)])
    emit(eq_idx, take, e_base, lambda c5: eqscr)

_sel_call = pl.kernel(
    _select_kernel,
    out_type=[
        jax.ShapeDtypeStruct((_B * _SELO,), jnp.float32),
        jax.ShapeDtypeStruct((_B * _SELO,), jnp.int32),
        jax.ShapeDtypeStruct((4 * _B * _SELO,), jnp.float32),
        jax.ShapeDtypeStruct((4 * _B * _SELO,), jnp.float32),
    ],
    mesh=plsc.VectorSubcoreMesh(
        core_axis_name="c", subcore_axis_name="s", num_cores=2
    ),
    compiler_params=pltpu.CompilerParams(needs_layout_passes=False),
    scratch_types=[
        pltpu.VMEM((_CHUNK * 2,), jnp.float32),
        pltpu.VMEM((_CHUNK,), jnp.uint32),
        pltpu.VMEM((16384,), jnp.int32),
        pltpu.VMEM((1024,), jnp.int32),
        pltpu.VMEM((1024,), jnp.int32),
        pltpu.VMEM((_CAP,), jnp.int32),
        pltpu.VMEM((_CAP,), jnp.float32),
        pltpu.VMEM((_CAP,), jnp.int32),
        pltpu.VMEM((128,), jnp.float32),
        pltpu.VMEM((128,), jnp.int32),
        pltpu.VMEM((128,), jnp.int32),
        pltpu.VMEM((128,), jnp.int32),
        pltpu.VMEM((16,), jnp.int32),
        pltpu.VMEM((256,), jnp.int32),
        pltpu.VMEM((_PERT,), jnp.float32),
        pltpu.VMEM((_PERT,), jnp.int32),
        pltpu.VMEM((128,), jnp.int32),
        pltpu.VMEM((128,), jnp.float32),
        pltpu.VMEM((128,), jnp.float32),
        pltpu.VMEM_SHARED((16384,), jnp.int32),
        pltpu.VMEM_SHARED((256,), jnp.int32),
        pltpu.SemaphoreType.DMA,
        pltpu.SemaphoreType.DMA,
    ],
)


def _nms_kernel(s_ref, tid_ref, a_ref, d_ref, out_ref, *, n_prop):
    s0 = s_ref[...]
    tid = tid_ref[...]
    a0 = a_ref[0]
    a1 = a_ref[1]
    a2 = a_ref[2]
    a3 = a_ref[3]
    d0 = d_ref[0] * 0.1
    d1 = d_ref[1] * 0.1
    d2 = d_ref[2] * 0.2
    d3 = d_ref[3] * 0.2
    h = a2 - a0
    w = a3 - a1
    cy = a0 + 0.5 * h + d0 * h
    cx = a1 + 0.5 * w + d1 * w
    hh = h * jnp.exp(d2)
    ww = w * jnp.exp(d3)
    y1 = cy - 0.5 * hh
    x1 = cx - 0.5 * ww
    y2 = y1 + hh
    x2 = x1 + ww
    one = jnp.float32(1.0)
    zero = jnp.float32(0.0)
    y1 = jnp.maximum(jnp.minimum(y1, one), zero)
    x1 = jnp.maximum(jnp.minimum(x1, one), zero)
    y2 = jnp.maximum(jnp.minimum(y2, one), zero)
    x2 = jnp.maximum(jnp.minimum(x2, one), zero)
    areas = (y2 - y1) * (x2 - x1)
    lane = jax.lax.broadcasted_iota(jnp.int32, (1, 128), 1)
    m0 = (lane == 0).astype(jnp.float32)
    m1 = (lane == 1).astype(jnp.float32)
    m2 = (lane == 2).astype(jnp.float32)
    m3 = (lane == 3).astype(jnp.float32)

    def step(t, s):
        m = jnp.max(s)
        tsel = jnp.min(jnp.where(s == m, tid, jnp.int32(2147483647)))
        pm = (s == m) & (tid == tsel)
        pmf = pm.astype(jnp.float32)
        py1 = jnp.sum(pmf * y1)
        px1 = jnp.sum(pmf * x1)
        py2 = jnp.sum(pmf * y2)
        px2 = jnp.sum(pmf * x2)
        pa = jnp.sum(pmf * areas)
        valid = (m > -1e8).astype(jnp.float32)
        row = (py1 * m0 + px1 * m1 + py2 * m2 + px2 * m3) * valid
        out_ref[pl.ds(t, 1), :] = row
        yy1 = jnp.maximum(py1, y1)
        xx1 = jnp.maximum(px1, x1)
        yy2 = jnp.minimum(py2, y2)
        xx2 = jnp.minimum(px2, x2)
        inter = jnp.maximum(yy2 - yy1, zero) * jnp.maximum(xx2 - xx1, zero)
        iou = inter / (pa + areas - inter + 1e-8)
        supp = (iou > _NMS_T) | pm
        return jnp.where(supp, jnp.float32(-1e9), s)

    jax.lax.fori_loop(0, n_prop, step, s0)


def _run_nms(s_p, tid_p, a_p, d_p, n_prop, out_rows, interpret=False):
    B, R, C = s_p.shape
    f = pl.pallas_call(
        functools.partial(_nms_kernel, n_prop=n_prop),
        grid=(B,),
        in_specs=[
            pl.BlockSpec((None, R, C), lambda b: (b, 0, 0)),
            pl.BlockSpec((None, R, C), lambda b: (b, 0, 0)),
            pl.BlockSpec((None, 4, R, C), lambda b: (b, 0, 0, 0)),
            pl.BlockSpec((None, 4, R, C), lambda b: (b, 0, 0, 0)),
        ],
        out_specs=pl.BlockSpec((None, out_rows, 128), lambda b: (b, 0, 0)),
        out_shape=jax.ShapeDtypeStruct((B, out_rows, 128), jnp.float32),
        interpret=interpret,
    )
    return f(s_p, tid_p, a_p, d_p)


def kernel(rpn_probs, rpn_bbox, anchors):
    B, N, _ = rpn_probs.shape
    probs_flat = jnp.pad(rpn_probs, ((0, 0), (0, _NPAD - N), (0, 0))).reshape(-1)
    bbox_flat = rpn_bbox.reshape(-1)
    anch_flat = anchors.reshape(-1)
    scr, idx, dlt, anc = _sel_call(probs_flat, bbox_flat, anch_flat)
    npts = _ROWS * _COLS
    s_p = scr.reshape(B, _SELO)[:, :npts].reshape(B, _ROWS, _COLS)
    tid_p = idx.reshape(B, _SELO)[:, :npts].reshape(B, _ROWS, _COLS)
    d_p = (
        dlt.reshape(4, B, _SELO)[:, :, :npts]
        .transpose(1, 0, 2)
        .reshape(B, 4, _ROWS, _COLS)
    )
    a_p = (
        anc.reshape(4, B, _SELO)[:, :, :npts]
        .transpose(1, 0, 2)
        .reshape(B, 4, _ROWS, _COLS)
    )
    out = _run_nms(s_p, tid_p, a_p, d_p, _PROPOSALS, 1024)
    return out[:, :_PROPOSALS, :4]


# speedup vs baseline: 1.0701x; 1.0701x over previous
import functools

import jax
import jax.numpy as jnp
from jax import lax
from jax.experimental import pallas as pl
from jax.experimental.pallas import tpu as pltpu
from jax.experimental.pallas import tpu_sc as plsc

_PROPOSALS = 1000
_NMS_T = 0.7
_PRE = 6000
_ROWS = 8
_COLS = 768

_B = 2
_N = 261888
_CHUNK = 16368
_NT = 16
_SELO = 6400
_CAP = 6016
_PERT = _SELO // _NT


def _select_kernel(
    probs_hbm, bbox_hbm, anch_hbm,
    scr_hbm, idx_hbm, dlt_hbm, anc_hbm,
    stage, ubuf, hist, merged, totals, abv_idx, abv_scr, eq_idx, eqscr,
    posbuf, posd, clbuf, cnt_pub, cnts_rd, initf, initi, idx4, rowd, rowa,
    spm_hist, spm_cnt, sem1, sem2,
):
    b = lax.axis_index("c")
    w = lax.axis_index("s")
    iota = lax.broadcasted_iota(jnp.int32, (16,), 0)
    zrow = jnp.zeros((16,), jnp.int32)

    neg = jnp.full((16,), -2e9, jnp.float32)

    def init_body(j, _):
        initf[pl.ds(j * 16, 16)] = neg
        initi[pl.ds(j * 16, 16)] = zrow
        return 0

    lax.fori_loop(0, _PERT // 16, init_body, 0)
    obase = b * _SELO + w * _PERT
    pltpu.sync_copy(initf, scr_hbm.at[pl.ds(obase, _PERT)])
    pltpu.sync_copy(initi, idx_hbm.at[pl.ds(obase, _PERT)])

    start = b * (_N * 2) + w * (_CHUNK * 2)
    pltpu.sync_copy(probs_hbm.at[pl.ds(start, _CHUNK * 2)], stage)
    io2 = iota * 2 + 1

    def deint(i, _):
        gi = plsc.load_gather(stage, [i * 32 + io2])
        ubuf[pl.ds(i * 16, 16)] = plsc.bitcast(gi, jnp.uint32)
        return 0

    lax.fori_loop(0, _CHUNK // 16, deint, 0)

    ones = jnp.ones((16,), jnp.int32)

    def hist_pass(shift, pshift, prefix, k):
        def z(j, _):
            hist[pl.ds(j * 16, 16)] = zrow
            return 0

        lax.fori_loop(0, 1024, z, 0)

        def scan(i, _):
            u = ubuf[pl.ds(i * 16, 16)]
            pred = jnp.right_shift(u, jnp.uint32(pshift)) == prefix
            digit = (jnp.right_shift(u, jnp.uint32(shift)) & jnp.uint32(1023)).astype(jnp.int32)
            plsc.addupdate_scatter(hist, [iota * 1024 + digit], ones, mask=pred)
            return 0

        lax.fori_loop(0, _CHUNK // 16, scan, 0)

        def mg(j, _):
            def inner(c, acc):
                return acc + hist[pl.ds(c * 1024 + j * 16, 16)]
            acc = lax.fori_loop(0, 16, inner, zrow)
            merged[pl.ds(j * 16, 16)] = acc
            return 0

        lax.fori_loop(0, 64, mg, 0)
        pltpu.sync_copy(merged, spm_hist.at[pl.ds(w * 1024, 1024)])
        plsc.subcore_barrier()
        pltpu.sync_copy(spm_hist, hist)
        plsc.subcore_barrier()

        def mg2(j, _):
            def inner(c, acc):
                return acc + hist[pl.ds(c * 1024 + j * 16, 16)]
            acc = lax.fori_loop(0, 16, inner, zrow)
            totals[pl.ds(j * 16, 16)] = acc
            return 0

        lax.fori_loop(0, 64, mg2, 0)

        def find(jj, carry):
            acc, found, bin_, rem = carry
            j = 63 - jj
            chunk = totals[pl.ds(j * 16, 16)]
            csum = jnp.sum(chunk)
            hit = jnp.logical_and(found == 0, acc + csum >= k)
            rv = lax.rev(chunk, (0,))
            cum = plsc.cumsum(rv)
            need = k - acc
            pos = jnp.sum(jnp.where(cum < need, 1, 0).astype(jnp.int32))
            cnt_at = jnp.sum(jnp.where(iota == pos, rv, 0))
            cum_at = jnp.sum(jnp.where(iota == pos, cum, 0))
            bin_new = j * 16 + 15 - pos
            rem_new = need - (cum_at - cnt_at)
            return (
                acc + jnp.where(hit, 0, csum),
                jnp.where(hit, jnp.int32(1), found),
                jnp.where(hit, bin_new, bin_),
                jnp.where(hit, rem_new, rem),
            )

        zero = jnp.int32(0)
        _, _, bin_, rem = lax.fori_loop(0, 64, find, (zero, zero, zero, zero))
        return bin_, rem

    b0, k1 = hist_pass(20, 30, jnp.uint32(0), jnp.int32(_PRE))
    b1, k2 = hist_pass(10, 20, b0.astype(jnp.uint32), k1)
    pref2 = ((b0 << 10) | b1).astype(jnp.uint32)
    b2, k_eq = hist_pass(0, 10, pref2, k2)
    t_bits = ((b0 << 20) | (b1 << 10) | b2).astype(jnp.uint32)
    t_vec = plsc.bitcast(jnp.zeros((16,), jnp.uint32) + t_bits, jnp.float32)

    def fill_eqscr(j, _):
        eqscr[pl.ds(j * 16, 16)] = t_vec
        return 0

    lax.fori_loop(0, 8, fill_eqscr, 0)

    base_g = w * _CHUNK

    def sel(i, carry):
        ac, ec = carry
        u = ubuf[pl.ds(i * 16, 16)]
        gidx = base_g + i * 16 + iota
        above = u > t_bits
        eq = u == t_bits
        am = above.astype(jnp.int32)
        em = eq.astype(jnp.int32)
        apos = ac + plsc.cumsum(am) - 1
        epos = ec + plsc.cumsum(em) - 1
        ma = jnp.logical_and(above, apos < _CAP)
        me = jnp.logical_and(eq, epos < _CAP)
        plsc.store_scatter(abv_idx, [apos], gidx, mask=ma)
        plsc.store_scatter(abv_scr, [apos], plsc.bitcast(u, jnp.float32), mask=ma)
        plsc.store_scatter(eq_idx, [epos], gidx, mask=me)
        return ac + jnp.sum(am), ec + jnp.sum(em)

    ac, ec = lax.fori_loop(0, _CHUNK // 16, sel, (jnp.int32(0), jnp.int32(0)))

    cnt_pub[pl.ds(0, 16)] = jnp.where(
        iota == 0, ac, jnp.where(iota == 1, ec, jnp.int32(0))
    )
    pltpu.sync_copy(cnt_pub, spm_cnt.at[pl.ds(w * 16, 16)])
    plsc.subcore_barrier()
    pltpu.sync_copy(spm_cnt, cnts_rd)
    plsc.subcore_barrier()

    def pref(x, carry):
        ap, ep, at = carry
        row = cnts_rd[pl.ds(x * 16, 16)]
        a_x = jnp.sum(jnp.where(iota == 0, row, 0))
        e_x = jnp.sum(jnp.where(iota == 1, row, 0))
        lt = (x < w).astype(jnp.int32)
        return ap + a_x * lt, ep + e_x * lt, at + a_x

    ap, ep, at = lax.fori_loop(
        0, 16, pref, (jnp.int32(0), jnp.int32(0), jnp.int32(0))
    )
    take = jnp.clip(k_eq - ep, 0, ec)
    a_base = b * _SELO + ap
    e_base = b * _SELO + at + jnp.minimum(ep, k_eq)
    trash = b * _SELO + 6144 + w * 16

    def zf(j, _):
        initf[pl.ds(j * 16, 16)] = jnp.zeros((16,), jnp.float32)
        return 0

    lax.fori_loop(0, _PERT // 16, zf, 0)
    for c in range(4):
        pltpu.sync_copy(initf, dlt_hbm.at[pl.ds(c * (_B * _SELO) + obase, _PERT)])
        pltpu.sync_copy(initf, anc_hbm.at[pl.ds(c * (_B * _SELO) + obase, _PERT)])

    def emit(src_idx, cnt, base, scr_src_fn):
        def chunk(c5, _):
            def fill(j, _):
                r = c5 * 128 + j * 16 + iota
                posbuf[pl.ds(j * 16, 16)] = jnp.where(r < cnt, base + r, trash)
                v = src_idx[pl.ds(c5 * 128 + j * 16, 16)]
                clbuf[pl.ds(j * 16, 16)] = (
                    jnp.minimum(jnp.maximum(v, 0), _N - 1) + b * _N
                ) * 4
                return 0

            lax.fori_loop(0, 8, fill, 0)
            d1 = pltpu.async_copy(
                src_idx.at[pl.ds(c5 * 128, 128)], idx_hbm.at[posbuf], sem1
            )
            d2 = pltpu.async_copy(scr_src_fn(c5), scr_hbm.at[posbuf], sem2)
            d1.wait()
            d2.wait()
            for c in range(4):
                def mk(j, _):
                    idx4[pl.ds(j * 16, 16)] = clbuf[pl.ds(j * 16, 16)] + c
                    posd[pl.ds(j * 16, 16)] = posbuf[pl.ds(j * 16, 16)] + c * (_B * _SELO)
                    return 0

                lax.fori_loop(0, 8, mk, 0)
                g1 = pltpu.async_copy(bbox_hbm.at[idx4], rowd, sem1)
                g2 = pltpu.async_copy(anch_hbm.at[idx4], rowa, sem2)
                g1.wait()
                g2.wait()
                s1 = pltpu.async_copy(rowd, dlt_hbm.at[posd], sem1)
                s2 = pltpu.async_copy(rowa, anc_hbm.at[posd], sem2)
                s1.wait()
                s2.wait()
            return 0

        lax.fori_loop(0, (cnt + 127) // 128, chunk, 0)

    emit(abv_idx, ac, a_base, lambda c5: abv_scr.at[pl.ds(c5 * 128, 128)])
    emit(eq_idx, take, e_base, lambda c5: eqscr)

_sel_call = pl.kernel(
    _select_kernel,
    out_type=[
        jax.ShapeDtypeStruct((_B * _SELO,), jnp.float32),
        jax.ShapeDtypeStruct((_B * _SELO,), jnp.int32),
        jax.ShapeDtypeStruct((4 * _B * _SELO,), jnp.float32),
        jax.ShapeDtypeStruct((4 * _B * _SELO,), jnp.float32),
    ],
    mesh=plsc.VectorSubcoreMesh(
        core_axis_name="c", subcore_axis_name="s", num_cores=2
    ),
    compiler_params=pltpu.CompilerParams(needs_layout_passes=False),
    scratch_types=[
        pltpu.VMEM((_CHUNK * 2,), jnp.float32),
        pltpu.VMEM((_CHUNK,), jnp.uint32),
        pltpu.VMEM((16384,), jnp.int32),
        pltpu.VMEM((1024,), jnp.int32),
        pltpu.VMEM((1024,), jnp.int32),
        pltpu.VMEM((_CAP,), jnp.int32),
        pltpu.VMEM((_CAP,), jnp.float32),
        pltpu.VMEM((_CAP,), jnp.int32),
        pltpu.VMEM((128,), jnp.float32),
        pltpu.VMEM((128,), jnp.int32),
        pltpu.VMEM((128,), jnp.int32),
        pltpu.VMEM((128,), jnp.int32),
        pltpu.VMEM((16,), jnp.int32),
        pltpu.VMEM((256,), jnp.int32),
        pltpu.VMEM((_PERT,), jnp.float32),
        pltpu.VMEM((_PERT,), jnp.int32),
        pltpu.VMEM((128,), jnp.int32),
        pltpu.VMEM((128,), jnp.float32),
        pltpu.VMEM((128,), jnp.float32),
        pltpu.VMEM_SHARED((16384,), jnp.int32),
        pltpu.VMEM_SHARED((256,), jnp.int32),
        pltpu.SemaphoreType.DMA,
        pltpu.SemaphoreType.DMA,
    ],
)


def _nms_kernel(s_ref, tid_ref, a_ref, d_ref, out_ref, *, n_prop):
    s0 = s_ref[...]
    tid = tid_ref[...]
    a0 = a_ref[0]
    a1 = a_ref[1]
    a2 = a_ref[2]
    a3 = a_ref[3]
    d0 = d_ref[0] * 0.1
    d1 = d_ref[1] * 0.1
    d2 = d_ref[2] * 0.2
    d3 = d_ref[3] * 0.2
    h = a2 - a0
    w = a3 - a1
    cy = a0 + 0.5 * h + d0 * h
    cx = a1 + 0.5 * w + d1 * w
    hh = h * jnp.exp(d2)
    ww = w * jnp.exp(d3)
    y1 = cy - 0.5 * hh
    x1 = cx - 0.5 * ww
    y2 = y1 + hh
    x2 = x1 + ww
    one = jnp.float32(1.0)
    zero = jnp.float32(0.0)
    y1 = jnp.maximum(jnp.minimum(y1, one), zero)
    x1 = jnp.maximum(jnp.minimum(x1, one), zero)
    y2 = jnp.maximum(jnp.minimum(y2, one), zero)
    x2 = jnp.maximum(jnp.minimum(x2, one), zero)
    areas = (y2 - y1) * (x2 - x1)
    lane = jax.lax.broadcasted_iota(jnp.int32, (1, 128), 1)
    m0 = (lane == 0).astype(jnp.float32)
    m1 = (lane == 1).astype(jnp.float32)
    m2 = (lane == 2).astype(jnp.float32)
    m3 = (lane == 3).astype(jnp.float32)

    def step(t, s):
        m = jnp.max(s)
        tsel = jnp.min(jnp.where(s == m, tid, jnp.int32(2147483647)))
        pm = (s == m) & (tid == tsel)
        pmf = pm.astype(jnp.float32)
        py1 = jnp.sum(pmf * y1)
        px1 = jnp.sum(pmf * x1)
        py2 = jnp.sum(pmf * y2)
        px2 = jnp.sum(pmf * x2)
        pa = jnp.sum(pmf * areas)
        valid = (m > -1e8).astype(jnp.float32)
        row = (py1 * m0 + px1 * m1 + py2 * m2 + px2 * m3) * valid
        out_ref[pl.ds(t, 1), :] = row
        yy1 = jnp.maximum(py1, y1)
        xx1 = jnp.maximum(px1, x1)
        yy2 = jnp.minimum(py2, y2)
        xx2 = jnp.minimum(px2, x2)
        inter = jnp.maximum(yy2 - yy1, zero) * jnp.maximum(xx2 - xx1, zero)
        iou = inter / (pa + areas - inter + 1e-8)
        supp = (iou > _NMS_T) | pm
        return jnp.where(supp, jnp.float32(-1e9), s)

    jax.lax.fori_loop(0, n_prop, step, s0)


def _run_nms(s_p, tid_p, a_p, d_p, n_prop, out_rows, interpret=False):
    B, R, C = s_p.shape
    f = pl.pallas_call(
        functools.partial(_nms_kernel, n_prop=n_prop),
        grid=(B,),
        in_specs=[
            pl.BlockSpec((None, R, C), lambda b: (b, 0, 0)),
            pl.BlockSpec((None, R, C), lambda b: (b, 0, 0)),
            pl.BlockSpec((None, 4, R, C), lambda b: (b, 0, 0, 0)),
            pl.BlockSpec((None, 4, R, C), lambda b: (b, 0, 0, 0)),
        ],
        out_specs=pl.BlockSpec((None, out_rows, 128), lambda b: (b, 0, 0)),
        out_shape=jax.ShapeDtypeStruct((B, out_rows, 128), jnp.float32),
        interpret=interpret,
    )
    return f(s_p, tid_p, a_p, d_p)


def kernel(rpn_probs, rpn_bbox, anchors):
    B, N, _ = rpn_probs.shape
    probs_flat = rpn_probs.reshape(-1)
    bbox_flat = rpn_bbox.reshape(-1)
    anch_flat = anchors.reshape(-1)
    scr, idx, dlt, anc = _sel_call(probs_flat, bbox_flat, anch_flat)
    npts = _ROWS * _COLS
    s_p = scr.reshape(B, _SELO)[:, :npts].reshape(B, _ROWS, _COLS)
    tid_p = idx.reshape(B, _SELO)[:, :npts].reshape(B, _ROWS, _COLS)
    d_p = (
        dlt.reshape(4, B, _SELO)[:, :, :npts]
        .transpose(1, 0, 2)
        .reshape(B, 4, _ROWS, _COLS)
    )
    a_p = (
        anc.reshape(4, B, _SELO)[:, :, :npts]
        .transpose(1, 0, 2)
        .reshape(B, 4, _ROWS, _COLS)
    )
    out = _run_nms(s_p, tid_p, a_p, d_p, _PROPOSALS, 1024)
    return out[:, :_PROPOSALS, :4]
